# baseline (speedup 1.0000x reference)
import functools

import jax
import jax.numpy as jnp
from jax import lax
from jax.experimental import pallas as pl
from jax.experimental.pallas import tpu as pltpu
from jax.experimental.pallas import tpu_sc as plsc

NC, NS = 2, 16
NW = NC * NS
CHUNK = 32
NBUF = 3


def _sc_gather(ids3, table, n_tokens):
    cpw = ids3.shape[1]
    H = table.shape[1]
    tok_per_w = cpw * CHUNK
    mesh = plsc.VectorSubcoreMesh(core_axis_name="c", subcore_axis_name="s")

    scratch = [pltpu.VMEM((cpw, CHUNK), jnp.int32)]
    scratch += [pltpu.VMEM((CHUNK, H), jnp.float32) for _ in range(NBUF)]
    scratch += [pltpu.SemaphoreType.DMA for _ in range(2 * NBUF)]

    @functools.partial(
        pl.kernel,
        mesh=mesh,
        out_type=jax.ShapeDtypeStruct((n_tokens, H), jnp.float32),
        scratch_types=scratch,
    )
    def k(ids_hbm, table_hbm, out_hbm, idx_v, *rest):
        bufs = rest[:NBUF]
        gsems = rest[NBUF:2 * NBUF]
        osems = rest[2 * NBUF:]
        wid = lax.axis_index("s") * NC + lax.axis_index("c")
        base = wid * tok_per_w
        pltpu.sync_copy(ids_hbm.at[wid], idx_v)

        gathers = [None] * NBUF
        outs = [None] * NBUF
        for ci in range(cpw):
            s = ci % NBUF
            if outs[s] is not None:
                outs[s].wait()
            gathers[s] = pltpu.async_copy(
                table_hbm.at[idx_v.at[ci]], bufs[s], gsems[s])
            p = ci - 1
            if p >= 0:
                sp = p % NBUF
                gathers[sp].wait()
                outs[sp] = pltpu.async_copy(
                    bufs[sp], out_hbm.at[pl.ds(base + p * CHUNK, CHUNK)],
                    osems[sp])
        last = cpw - 1
        sl = last % NBUF
        gathers[sl].wait()
        outs[sl] = pltpu.async_copy(
            bufs[sl], out_hbm.at[pl.ds(base + last * CHUNK, CHUNK)], osems[sl])
        for s in range(NBUF):
            if outs[s] is not None:
                outs[s].wait()

    return k(ids3, table)


def _tc_body(emb_ref, gamma_ref, beta_ref, wg_ref, sg_ref, c_ref,
             w2_ref, b2_ref, _a0, _a1, normed_ref, logits_ref):
    x = emb_ref[...]
    rH = 1.0 / x.shape[1]
    mu = jnp.sum(x, axis=1, keepdims=True) * rH
    var = jnp.sum(x * x, axis=1, keepdims=True) * rH - mu * mu
    inv = lax.rsqrt(var + 1e-5)
    xb = x.astype(jnp.bfloat16)
    hraw = jnp.dot(xb, wg_ref[...], preferred_element_type=jnp.float32)
    normed_ref[...] = (x - mu) * inv * gamma_ref[...] + beta_ref[...]
    h = jnp.maximum(hraw * inv - (inv * mu) * sg_ref[...] + c_ref[...], 0.0)
    logits_ref[...] = jnp.dot(h.astype(jnp.bfloat16), w2_ref[...],
                              preferred_element_type=jnp.float32) + b2_ref[...]


def _tc_head(emb, gamma, beta, Wg, sg, c, W2, b2, blk, row0, n_total,
             normed_prev, logits_prev):
    ns, H = emb.shape
    Hh = Wg.shape[1]
    C = W2.shape[1]
    grid = (ns // blk,)
    off = row0 // blk
    full = lambda shape: pl.BlockSpec(shape, lambda i: (0, 0))
    any_spec = pl.BlockSpec(memory_space=pl.ANY)
    if normed_prev is None:
        normed_prev = jnp.zeros((8, 128), jnp.float32)
        logits_prev = jnp.zeros((8, 128), jnp.float32)
        aliases = {}
    else:
        aliases = {8: 0, 9: 1}
    normed, logits = pl.pallas_call(
        _tc_body,
        grid=grid,
        in_specs=[
            pl.BlockSpec((blk, H), lambda i: (i, 0)),
            full((1, H)), full((1, H)),
            full((H, Hh)), full((1, Hh)), full((1, Hh)),
            full((Hh, C)), full((1, C)),
            any_spec, any_spec,
        ],
        out_specs=[
            pl.BlockSpec((blk, H), lambda i: (off + i, 0)),
            pl.BlockSpec((blk, C), lambda i: (off + i, 0)),
        ],
        out_shape=[
            jax.ShapeDtypeStruct((n_total, H), jnp.float32),
            jax.ShapeDtypeStruct((n_total, C), jnp.float32),
        ],
        input_output_aliases=aliases,
    )(emb, gamma.reshape(1, H), beta.reshape(1, H),
      Wg, sg, c, W2, b2.reshape(1, C),
      normed_prev, logits_prev)
    return normed, logits


SLICES = (16384,)


def kernel(input_ids, table, gamma, beta, W1, b1, W2, b2):
    B, S = input_ids.shape
    V, H = table.shape
    n = B * S
    C = W2.shape[1]
    flat = input_ids.reshape(n).astype(jnp.int32)
    Hh = W1.shape[1]
    wg_f32 = gamma[:, None] * W1
    wg = wg_f32.astype(jnp.bfloat16)
    sg = jnp.sum(wg_f32, axis=0).reshape(1, Hh)
    c = (beta @ W1 + b1).reshape(1, Hh)
    w2b = W2.astype(jnp.bfloat16)
    embs = []
    row0 = 0
    for ns in SLICES:
        ids_s = lax.slice(flat, (row0,), (row0 + ns,)).reshape(
            NW, ns // (NW * CHUNK), CHUNK)
        embs.append(_sc_gather(ids_s, table, ns))
        row0 += ns
    normed, logits = None, None
    row0 = 0
    for emb, ns in zip(embs, SLICES):
        normed, logits = _tc_head(emb, gamma, beta, wg, sg, c, w2b, b2,
                                  blk=2048, row0=row0, n_total=n,
                                  normed_prev=normed, logits_prev=logits)
        row0 += ns
    return normed.reshape(B, S, H), logits.reshape(B, S, C)


# speedup vs baseline: 1.0153x; 1.0153x over previous
import functools

import jax
import jax.numpy as jnp
from jax import lax
from jax.experimental import pallas as pl
from jax.experimental.pallas import tpu as pltpu
from jax.experimental.pallas import tpu_sc as plsc

NC, NS = 2, 16
NW = NC * NS
CHUNK = 32
NBUF = 3


def _sc_gather(ids3, table, n_tokens):
    cpw = ids3.shape[1]
    H = table.shape[1]
    tok_per_w = cpw * CHUNK
    mesh = plsc.VectorSubcoreMesh(core_axis_name="c", subcore_axis_name="s")

    scratch = [pltpu.VMEM((cpw, CHUNK), jnp.int32)]
    scratch += [pltpu.VMEM((CHUNK, H), jnp.float32) for _ in range(NBUF)]
    scratch += [pltpu.SemaphoreType.DMA for _ in range(2 * NBUF)]

    @functools.partial(
        pl.kernel,
        mesh=mesh,
        out_type=jax.ShapeDtypeStruct((n_tokens, H), jnp.float32),
        scratch_types=scratch,
    )
    def k(ids_hbm, table_hbm, out_hbm, idx_v, *rest):
        bufs = rest[:NBUF]
        gsems = rest[NBUF:2 * NBUF]
        osems = rest[2 * NBUF:]
        wid = lax.axis_index("s") * NC + lax.axis_index("c")
        base = wid * tok_per_w
        pltpu.sync_copy(ids_hbm.at[wid], idx_v)

        gathers = [None] * NBUF
        outs = [None] * NBUF
        for ci in range(cpw):
            s = ci % NBUF
            if outs[s] is not None:
                outs[s].wait()
            gathers[s] = pltpu.async_copy(
                table_hbm.at[idx_v.at[ci]], bufs[s], gsems[s])
            p = ci - 1
            if p >= 0:
                sp = p % NBUF
                gathers[sp].wait()
                outs[sp] = pltpu.async_copy(
                    bufs[sp], out_hbm.at[pl.ds(base + p * CHUNK, CHUNK)],
                    osems[sp])
        last = cpw - 1
        sl = last % NBUF
        gathers[sl].wait()
        outs[sl] = pltpu.async_copy(
            bufs[sl], out_hbm.at[pl.ds(base + last * CHUNK, CHUNK)], osems[sl])
        for s in range(NBUF):
            if outs[s] is not None:
                outs[s].wait()

    return k(ids3, table)


def _tc_body(emb_ref, gamma_ref, beta_ref, w1_ref, b1_ref, w2_ref, b2_ref,
             _a0, _a1, normed_ref, logits_ref):
    x = emb_ref[...]
    rH = 1.0 / x.shape[1]
    mu = jnp.sum(x, axis=1, keepdims=True) * rH
    var = jnp.sum(x * x, axis=1, keepdims=True) * rH - mu * mu
    inv = lax.rsqrt(var + 1e-5)
    normed = (x - mu) * inv * gamma_ref[...] + beta_ref[...]
    normed_ref[...] = normed
    h = jnp.dot(normed.astype(jnp.bfloat16), w1_ref[...],
                preferred_element_type=jnp.float32) + b1_ref[...]
    h = jnp.maximum(h, 0.0)
    logits_ref[...] = jnp.dot(h.astype(jnp.bfloat16), w2_ref[...],
                              preferred_element_type=jnp.float32) + b2_ref[...]


def _tc_head(emb, gamma, beta, W1, b1, W2, b2, blk, row0, n_total,
             normed_prev, logits_prev):
    ns, H = emb.shape
    Hh = W1.shape[1]
    C = W2.shape[1]
    grid = (ns // blk,)
    off = row0 // blk
    full = lambda shape: pl.BlockSpec(shape, lambda i: (0, 0))
    any_spec = pl.BlockSpec(memory_space=pl.ANY)
    if normed_prev is None:
        normed_prev = jnp.zeros((8, 128), jnp.float32)
        logits_prev = jnp.zeros((8, 128), jnp.float32)
        aliases = {}
    else:
        aliases = {7: 0, 8: 1}
    normed, logits = pl.pallas_call(
        _tc_body,
        grid=grid,
        in_specs=[
            pl.BlockSpec((blk, H), lambda i: (i, 0)),
            full((1, H)), full((1, H)),
            full((H, Hh)), full((1, Hh)),
            full((Hh, C)), full((1, C)),
            any_spec, any_spec,
        ],
        out_specs=[
            pl.BlockSpec((blk, H), lambda i: (off + i, 0)),
            pl.BlockSpec((blk, C), lambda i: (off + i, 0)),
        ],
        out_shape=[
            jax.ShapeDtypeStruct((n_total, H), jnp.float32),
            jax.ShapeDtypeStruct((n_total, C), jnp.float32),
        ],
        input_output_aliases=aliases,
    )(emb, gamma.reshape(1, H), beta.reshape(1, H),
      W1, b1.reshape(1, Hh), W2, b2.reshape(1, C),
      normed_prev, logits_prev)
    return normed, logits


SLICES = (16384,)


def kernel(input_ids, table, gamma, beta, W1, b1, W2, b2):
    B, S = input_ids.shape
    V, H = table.shape
    n = B * S
    C = W2.shape[1]
    flat = input_ids.reshape(n).astype(jnp.int32)
    w1b = W1.astype(jnp.bfloat16)
    w2b = W2.astype(jnp.bfloat16)
    embs = []
    row0 = 0
    for ns in SLICES:
        ids_s = lax.slice(flat, (row0,), (row0 + ns,)).reshape(
            NW, ns // (NW * CHUNK), CHUNK)
        embs.append(_sc_gather(ids_s, table, ns))
        row0 += ns
    normed, logits = None, None
    row0 = 0
    for emb, ns in zip(embs, SLICES):
        normed, logits = _tc_head(emb, gamma, beta, w1b, b1, w2b, b2,
                                  blk=1024, row0=row0, n_total=n,
                                  normed_prev=normed, logits_prev=logits)
        row0 += ns
    return normed.reshape(B, S, H), logits.reshape(B, S, C)


# speedup vs baseline: 1.0249x; 1.0094x over previous
import functools

import jax
import jax.numpy as jnp
from jax import lax
from jax.experimental import pallas as pl
from jax.experimental.pallas import tpu as pltpu
from jax.experimental.pallas import tpu_sc as plsc

NC, NS = 2, 16
NW = NC * NS
CHUNK = 32
NBUF = 3


def _sc_gather(ids3, table, n_tokens):
    cpw = ids3.shape[1]
    H = table.shape[1]
    tok_per_w = cpw * CHUNK
    mesh = plsc.VectorSubcoreMesh(core_axis_name="c", subcore_axis_name="s")

    scratch = [pltpu.VMEM((cpw, CHUNK), jnp.int32)]
    scratch += [pltpu.VMEM((CHUNK, H), jnp.float32) for _ in range(NBUF)]
    scratch += [pltpu.SemaphoreType.DMA for _ in range(2 * NBUF)]

    @functools.partial(
        pl.kernel,
        mesh=mesh,
        out_type=jax.ShapeDtypeStruct((n_tokens, H), jnp.float32),
        scratch_types=scratch,
    )
    def k(ids_hbm, table_hbm, out_hbm, idx_v, *rest):
        bufs = rest[:NBUF]
        gsems = rest[NBUF:2 * NBUF]
        osems = rest[2 * NBUF:]
        wid = lax.axis_index("s") * NC + lax.axis_index("c")
        base = wid * tok_per_w
        pltpu.sync_copy(ids_hbm.at[wid], idx_v)

        gathers = [None] * NBUF
        outs = [None] * NBUF
        for ci in range(cpw):
            s = ci % NBUF
            if outs[s] is not None:
                outs[s].wait()
            gathers[s] = pltpu.async_copy(
                table_hbm.at[idx_v.at[ci]], bufs[s], gsems[s])
            p = ci - 1
            if p >= 0:
                sp = p % NBUF
                gathers[sp].wait()
                outs[sp] = pltpu.async_copy(
                    bufs[sp], out_hbm.at[pl.ds(base + p * CHUNK, CHUNK)],
                    osems[sp])
        last = cpw - 1
        sl = last % NBUF
        gathers[sl].wait()
        outs[sl] = pltpu.async_copy(
            bufs[sl], out_hbm.at[pl.ds(base + last * CHUNK, CHUNK)], osems[sl])
        for s in range(NBUF):
            if outs[s] is not None:
                outs[s].wait()

    return k(ids3, table)


def _tc_body(emb_ref, gamma_ref, beta_ref, w1_ref, b1_ref, w2_ref, b2_ref,
             _a0, _a1, normed_ref, logits_ref):
    x = emb_ref[...]
    rH = 1.0 / x.shape[1]
    mu = jnp.sum(x, axis=1, keepdims=True) * rH
    var = jnp.sum(x * x, axis=1, keepdims=True) * rH - mu * mu
    inv = lax.rsqrt(var + 1e-5)
    normed = (x - mu) * inv * gamma_ref[...] + beta_ref[...]
    normed_ref[...] = normed
    h = jnp.dot(normed.astype(jnp.bfloat16), w1_ref[...],
                preferred_element_type=jnp.float32) + b1_ref[...]
    h = jnp.maximum(h, 0.0)
    logits_ref[...] = jnp.dot(h.astype(jnp.bfloat16), w2_ref[...],
                              preferred_element_type=jnp.float32) + b2_ref[...]


def _tc_head(emb, gamma, beta, W1, b1, W2, b2, blk, row0, n_total,
             normed_prev, logits_prev):
    ns, H = emb.shape
    Hh = W1.shape[1]
    C = W2.shape[1]
    grid = (ns // blk,)
    off = row0 // blk
    full = lambda shape: pl.BlockSpec(shape, lambda i: (0, 0))
    any_spec = pl.BlockSpec(memory_space=pl.ANY)
    if normed_prev is None:
        normed_prev = jnp.zeros((8, 128), jnp.float32)
        logits_prev = jnp.zeros((8, 128), jnp.float32)
        aliases = {}
    else:
        aliases = {7: 0, 8: 1}
    normed, logits = pl.pallas_call(
        _tc_body,
        grid=grid,
        in_specs=[
            pl.BlockSpec((blk, H), lambda i: (i, 0)),
            full((1, H)), full((1, H)),
            full((H, Hh)), full((1, Hh)),
            full((Hh, C)), full((1, C)),
            any_spec, any_spec,
        ],
        out_specs=[
            pl.BlockSpec((blk, H), lambda i: (off + i, 0)),
            pl.BlockSpec((blk, C), lambda i: (off + i, 0)),
        ],
        out_shape=[
            jax.ShapeDtypeStruct((n_total, H), jnp.float32),
            jax.ShapeDtypeStruct((n_total, C), jnp.float32),
        ],
        input_output_aliases=aliases,
    )(emb, gamma.reshape(1, H), beta.reshape(1, H),
      W1, b1.reshape(1, Hh), W2, b2.reshape(1, C),
      normed_prev, logits_prev)
    return normed, logits


SLICES = (16384,)


def kernel(input_ids, table, gamma, beta, W1, b1, W2, b2):
    B, S = input_ids.shape
    V, H = table.shape
    n = B * S
    C = W2.shape[1]
    flat = input_ids.reshape(n).astype(jnp.int32)
    w1b = W1.astype(jnp.bfloat16)
    w2b = W2.astype(jnp.bfloat16)
    embs = []
    row0 = 0
    for ns in SLICES:
        ids_s = lax.slice(flat, (row0,), (row0 + ns,)).reshape(
            NW, ns // (NW * CHUNK), CHUNK)
        embs.append(_sc_gather(ids_s, table, ns))
        row0 += ns
    normed, logits = None, None
    row0 = 0
    for emb, ns in zip(embs, SLICES):
        normed, logits = _tc_head(emb, gamma, beta, w1b, b1, w2b, b2,
                                  blk=2048, row0=row0, n_total=n,
                                  normed_prev=normed, logits_prev=logits)
        row0 += ns
    return normed.reshape(B, S, H), logits.reshape(B, S, C)


# speedup vs baseline: 1.0319x; 1.0068x over previous
import functools

import jax
import jax.numpy as jnp
from jax import lax
from jax.experimental import pallas as pl
from jax.experimental.pallas import tpu as pltpu
from jax.experimental.pallas import tpu_sc as plsc

NC, NS = 2, 16
NW = NC * NS
CHUNK = 32
NBUF = 3


def _sc_gather(ids3, table, n_tokens):
    cpw = ids3.shape[1]
    H = table.shape[1]
    tok_per_w = cpw * CHUNK
    mesh = plsc.VectorSubcoreMesh(core_axis_name="c", subcore_axis_name="s")

    scratch = [pltpu.VMEM((cpw, CHUNK), jnp.int32)]
    scratch += [pltpu.VMEM((CHUNK, H), jnp.float32) for _ in range(NBUF)]
    scratch += [pltpu.SemaphoreType.DMA for _ in range(2 * NBUF)]

    @functools.partial(
        pl.kernel,
        mesh=mesh,
        out_type=jax.ShapeDtypeStruct((n_tokens, H), jnp.float32),
        scratch_types=scratch,
    )
    def k(ids_hbm, table_hbm, out_hbm, idx_v, *rest):
        bufs = rest[:NBUF]
        gsems = rest[NBUF:2 * NBUF]
        osems = rest[2 * NBUF:]
        wid = lax.axis_index("s") * NC + lax.axis_index("c")
        base = wid * tok_per_w
        pltpu.sync_copy(ids_hbm.at[wid], idx_v)

        gathers = [None] * NBUF
        outs = [None] * NBUF
        for ci in range(cpw):
            s = ci % NBUF
            if outs[s] is not None:
                outs[s].wait()
            gathers[s] = pltpu.async_copy(
                table_hbm.at[idx_v.at[ci]], bufs[s], gsems[s])
            p = ci - 1
            if p >= 0:
                sp = p % NBUF
                gathers[sp].wait()
                outs[sp] = pltpu.async_copy(
                    bufs[sp], out_hbm.at[pl.ds(base + p * CHUNK, CHUNK)],
                    osems[sp])
        last = cpw - 1
        sl = last % NBUF
        gathers[sl].wait()
        outs[sl] = pltpu.async_copy(
            bufs[sl], out_hbm.at[pl.ds(base + last * CHUNK, CHUNK)], osems[sl])
        for s in range(NBUF):
            if outs[s] is not None:
                outs[s].wait()

    return k(ids3, table)


def _tc_body(emb_ref, gamma_ref, beta_ref, w1_ref, b1_ref, w2_ref, b2_ref,
             normed_ref, logits_ref):
    x = emb_ref[...]
    rH = 1.0 / x.shape[1]
    mu = jnp.sum(x, axis=1, keepdims=True) * rH
    var = jnp.sum(x * x, axis=1, keepdims=True) * rH - mu * mu
    inv = lax.rsqrt(var + 1e-5)
    normed = (x - mu) * inv * gamma_ref[...] + beta_ref[...]
    normed_ref[...] = normed
    h = jnp.dot(normed.astype(jnp.bfloat16), w1_ref[...],
                preferred_element_type=jnp.float32) + b1_ref[...]
    h = jnp.maximum(h, 0.0)
    logits_ref[...] = jnp.dot(h.astype(jnp.bfloat16), w2_ref[...],
                              preferred_element_type=jnp.float32) + b2_ref[...]


def _tc_head(emb, gamma, beta, W1, b1, W2, b2, blk):
    n, H = emb.shape
    Hh = W1.shape[1]
    C = W2.shape[1]
    full = lambda shape: pl.BlockSpec(shape, lambda i: (0, 0))
    return pl.pallas_call(
        _tc_body,
        grid=(n // blk,),
        in_specs=[
            pl.BlockSpec((blk, H), lambda i: (i, 0)),
            full((1, H)), full((1, H)),
            full((H, Hh)), full((1, Hh)),
            full((Hh, C)), full((1, C)),
        ],
        out_specs=[
            pl.BlockSpec((blk, H), lambda i: (i, 0)),
            pl.BlockSpec((blk, C), lambda i: (i, 0)),
        ],
        out_shape=[
            jax.ShapeDtypeStruct((n, H), jnp.float32),
            jax.ShapeDtypeStruct((n, C), jnp.float32),
        ],
    )(emb, gamma.reshape(1, H), beta.reshape(1, H),
      W1, b1.reshape(1, Hh), W2, b2.reshape(1, C))


def kernel(input_ids, table, gamma, beta, W1, b1, W2, b2):
    B, S = input_ids.shape
    V, H = table.shape
    n = B * S
    C = W2.shape[1]
    ids = input_ids.reshape(NW, n // (NW * CHUNK), CHUNK).astype(jnp.int32)
    emb = _sc_gather(ids, table, n)
    normed, logits = _tc_head(emb, gamma, beta, W1.astype(jnp.bfloat16), b1,
                              W2.astype(jnp.bfloat16), b2, blk=2048)
    return normed.reshape(B, S, H), logits.reshape(B, S, C)
